# baseline (speedup 1.0000x reference)
import functools

import jax
import jax.numpy as jnp
from jax import lax
from jax.experimental import pallas as pl
from jax.experimental.pallas import tpu as pltpu
from jax.experimental.pallas import tpu_sc as plsc

D_MODEL = 64
SCALE = 8.0

NC = 2
NS = 16
NW = NC * NS
LANES = 16

CHUNK = 1024
SUB = 128
N_SUB = CHUNK // SUB


@functools.partial(jax.jit, static_argnames=("n_rows",))
def _embed_lookup(x_flat2d, table, n_rows):
    b_per_w = n_rows // NW
    n_chunks = b_per_w // CHUNK
    mesh = plsc.VectorSubcoreMesh(
        core_axis_name="c", subcore_axis_name="s", num_cores=NC, num_subcores=NS
    )

    @functools.partial(
        pl.kernel,
        out_type=jax.ShapeDtypeStruct((n_rows, D_MODEL), jnp.float32),
        mesh=mesh,
        scratch_types=[
            pltpu.VMEM((N_SUB, SUB), jnp.int32),
            pltpu.VMEM((CHUNK, D_MODEL), jnp.float32),
            pltpu.SemaphoreType.DMA,
        ],
        compiler_params=pltpu.CompilerParams(use_tc_tiling_on_sc=False),
    )
    def body(x_hbm, table_hbm, out_hbm, idx_v, rows_v, sem):
        wid = lax.axis_index("s") * NC + lax.axis_index("c")
        base = wid * b_per_w

        def do_chunk(g, _):
            row0 = pl.multiple_of(base + g * CHUNK, CHUNK)
            pltpu.sync_copy(
                x_hbm.at[pl.ds(pl.multiple_of(row0 // SUB, 8), N_SUB)], idx_v
            )
            copies = []
            for j in range(N_SUB):
                copies.append(
                    pltpu.async_copy(
                        table_hbm.at[idx_v.at[j]],
                        rows_v.at[pl.ds(j * SUB, SUB)],
                        sem,
                    )
                )
            for c in copies:
                c.wait()

            def scale_row(r, _):
                for j in range(D_MODEL // LANES):
                    sl = pl.ds(j * LANES, LANES)
                    rows_v[r, sl] = rows_v[r, sl] * SCALE
                return ()

            lax.fori_loop(0, CHUNK, scale_row, (), unroll=4)

            pltpu.sync_copy(rows_v, out_hbm.at[pl.ds(row0, CHUNK)])
            return ()

        lax.fori_loop(0, n_chunks, do_chunk, ())

    return body(x_flat2d, table)


def kernel(x, table):
    b, s = x.shape
    n_rows = b * s
    x_flat2d = x.reshape(n_rows // SUB, SUB).astype(jnp.int32)
    out = _embed_lookup(x_flat2d, table, n_rows)
    return out.reshape(b, s, D_MODEL)
